# baseline (speedup 1.0000x reference)
import jax
import jax.numpy as jnp
from jax.experimental import pallas as pl
from jax.experimental.pallas import tpu as pltpu

_F32 = jnp.float32
_BF16 = jnp.bfloat16


def _bdot(a, b):
    return jnp.dot(a.astype(_BF16), b.astype(_BF16),
                   preferred_element_type=_F32)


def _ln(h, g, b):
    m = jnp.mean(h, axis=-1, keepdims=True)
    v = jnp.mean((h - m) * (h - m), axis=-1, keepdims=True)
    return (h - m) * jax.lax.rsqrt(v + 1e-5) * g + b


def _shift_dn(x):
    z = jnp.zeros((1, x.shape[1]), x.dtype)
    return jnp.concatenate([z, x[:-1, :]], axis=0)


def _shift_up(x):
    z = jnp.zeros((1, x.shape[1]), x.dtype)
    return jnp.concatenate([x[1:, :], z], axis=0)


def _vp_body(h, w1, b1, g1, be1, w2, b2, g2, be2, lwv, lb):
    x3 = jnp.concatenate([_shift_dn(h), h, _shift_up(h)], axis=1)
    h = jax.nn.relu(_bdot(x3, w1) + b1)
    h = _ln(h, g1, be1)
    x3 = jnp.concatenate([_shift_dn(h), h, _shift_up(h)], axis=1)
    h = jax.nn.relu(_bdot(x3, w2) + b2)
    h = _ln(h, g2, be2)
    return _bdot(h, lwv) + lb


def _mega_kernel(S, T, D, C, NB,
                 x_ref, durf_ref, ptrg_ref, bins_ref, maxd_ref,
                 dw1, db1, dg1, dbe1, dw2, db2, dg2, dbe2, dlw, dlb,
                 pw1, pb1, pg1, pbe1, pw2, pb2, pg2, pbe2, plw, plb,
                 ew1, eb1, eg1, ebe1, ew2, eb2, eg2, ebe2, elw, elb,
                 ptab_ref, etab_ref,
                 xe_ref, pemb_ref, eemb_ref, ppred_ref, epred_ref,
                 dpred_ref, maskf_ref):
    xb = x_ref[0]
    durf = durf_ref[0]
    ptrg = ptrg_ref[0]
    maxd = maxd_ref[0, 0]

    dpred = _vp_body(xb, dw1[...], db1[...], dg1[...], dbe1[...],
                     dw2[...], db2[...], dg2[...], dbe2[...], dlw[...], dlb[...])
    dpred_ref[0, 0, :] = dpred[:, 0]

    r = jax.lax.broadcasted_iota(jnp.int32, (S, S), 0)
    c = jax.lax.broadcasted_iota(jnp.int32, (S, S), 1)
    tri = (r <= c).astype(_F32)
    cum = _bdot(durf, tri)
    cum_prev = cum - durf
    mel_len = cum[0, S - 1]
    lim = jnp.minimum(mel_len, maxd)

    tt = jax.lax.broadcasted_iota(jnp.int32, (T, 1), 0).astype(_F32)
    validf = (tt < lim).astype(_F32)
    E = ((cum_prev <= tt) & (tt < cum)).astype(_F32)
    E = E * validf
    xe0 = _bdot(E, xb)
    maskf_ref[0, 0, :] = 1.0 - validf[:, 0]

    pv = ptrg.reshape(T, 1)
    binsrow = bins_ref[0:1, :]
    cnt = jnp.sum((binsrow < pv).astype(_F32), axis=1, keepdims=True)
    nn = jax.lax.broadcasted_iota(jnp.int32, (T, NB), 1).astype(_F32)
    onehot = (nn == cnt).astype(_F32)
    pemb = _bdot(onehot, ptab_ref[...])
    eemb = _bdot(onehot, etab_ref[...])
    pemb_ref[0] = pemb
    eemb_ref[0] = eemb

    ppred = _vp_body(xe0, pw1[...], pb1[...], pg1[...], pbe1[...],
                     pw2[...], pb2[...], pg2[...], pbe2[...], plw[...], plb[...])
    ppred_ref[0, 0, :] = ppred[:, 0] * validf[:, 0]
    xe1 = xe0 + pemb

    epred = _vp_body(xe1, ew1[...], eb1[...], eg1[...], ebe1[...],
                     ew2[...], eb2[...], eg2[...], ebe2[...], elw[...], elb[...])
    epred_ref[0, 0, :] = epred[:, 0] * validf[:, 0]
    xe_ref[0] = xe1 + eemb


def _flat_conv_w(w):
    K = w.shape[2]
    return w.transpose(2, 1, 0).reshape(K * w.shape[1], w.shape[0])


def _vp_args(p):
    C = p['c1b'].shape[0]
    return (
        _flat_conv_w(p['c1w']), p['c1b'].reshape(1, C),
        p['g1'].reshape(1, C), p['b1'].reshape(1, C),
        _flat_conv_w(p['c2w']), p['c2b'].reshape(1, C),
        p['g2'].reshape(1, C), p['b2'].reshape(1, C),
        p['lw'].reshape(C, 1), p['lb'].reshape(1, 1),
    )


def kernel(x, dur_trg, pitch_trg, energy_trg, src_mask, max_dur,
           dp, pp, ep, pitch_bins, energy_bins, pitch_table, energy_table):
    B, S, D = x.shape
    T = pitch_trg.shape[1]
    C = dp['c1b'].shape[0]
    NB = pitch_table.shape[0]

    durf = dur_trg.astype(_F32).reshape(B, 1, S)
    ptrg = pitch_trg.reshape(B, 1, T)
    bins_pad = jnp.concatenate(
        [pitch_bins.astype(_F32), jnp.full((NB - pitch_bins.shape[0],), 1e30, _F32)]
    ).reshape(1, NB)
    bins_pad = jnp.broadcast_to(bins_pad, (8, NB))
    maxd_arr = jnp.full((8, 128), max_dur, _F32)

    vp_all = _vp_args(dp) + _vp_args(pp) + _vp_args(ep)

    def full(a):
        return pl.BlockSpec(a.shape, lambda b: (0,) * a.ndim)

    in_specs = (
        [pl.BlockSpec((1, S, D), lambda b: (b, 0, 0)),
         pl.BlockSpec((1, 1, S), lambda b: (b, 0, 0)),
         pl.BlockSpec((1, 1, T), lambda b: (b, 0, 0)),
         full(bins_pad), full(maxd_arr)]
        + [full(a) for a in vp_all]
        + [full(pitch_table), full(energy_table)]
    )
    out_specs = [
        pl.BlockSpec((1, T, D), lambda b: (b, 0, 0)),
        pl.BlockSpec((1, T, D), lambda b: (b, 0, 0)),
        pl.BlockSpec((1, T, D), lambda b: (b, 0, 0)),
        pl.BlockSpec((1, 1, T), lambda b: (b, 0, 0)),
        pl.BlockSpec((1, 1, T), lambda b: (b, 0, 0)),
        pl.BlockSpec((1, 1, S), lambda b: (b, 0, 0)),
        pl.BlockSpec((1, 1, T), lambda b: (b, 0, 0)),
    ]
    out_shapes = [
        jax.ShapeDtypeStruct((B, T, D), _F32),
        jax.ShapeDtypeStruct((B, T, D), _F32),
        jax.ShapeDtypeStruct((B, T, D), _F32),
        jax.ShapeDtypeStruct((B, 1, T), _F32),
        jax.ShapeDtypeStruct((B, 1, T), _F32),
        jax.ShapeDtypeStruct((B, 1, S), _F32),
        jax.ShapeDtypeStruct((B, 1, T), _F32),
    ]

    import functools
    body = functools.partial(_mega_kernel, S, T, D, C, NB)
    xe, pemb, eemb, ppred, epred, dpred, maskf = pl.pallas_call(
        body,
        grid=(B,),
        in_specs=in_specs,
        out_specs=out_specs,
        out_shape=out_shapes,
    )(x, durf, ptrg, bins_pad, maxd_arr, *vp_all, pitch_table, energy_table)

    mel_mask = maskf.reshape(B, T) > 0.5
    log_dur_pred = jnp.where(src_mask, 0.0, dpred.reshape(B, S))
    return (xe, mel_mask, log_dur_pred, dur_trg,
            ppred.reshape(B, T), pemb, epred.reshape(B, T), eemb)


# speedup vs baseline: 1.3201x; 1.3201x over previous
import functools

import jax
import jax.numpy as jnp
from jax.experimental import pallas as pl

_F32 = jnp.float32
_BF16 = jnp.bfloat16


def _bdot(a, b):
    return jnp.dot(a.astype(_BF16), b.astype(_BF16),
                   preferred_element_type=_F32)


def _row_dot(vrow, m):
    return jax.lax.dot_general(
        vrow.astype(_BF16), m.astype(_BF16),
        (((1,), (1,)), ((), ())), preferred_element_type=_F32)


def _ln(h, g, b):
    m = jnp.mean(h, axis=-1, keepdims=True)
    v = jnp.mean((h - m) * (h - m), axis=-1, keepdims=True)
    return (h - m) * jax.lax.rsqrt(v + 1e-5) * g + b


def _shift_dn(x):
    z = jnp.zeros((1, x.shape[1]), x.dtype)
    return jnp.concatenate([z, x[:-1, :]], axis=0)


def _shift_up(x):
    z = jnp.zeros((1, x.shape[1]), x.dtype)
    return jnp.concatenate([x[1:, :], z], axis=0)


def _conv3(h, wwide, b, C):
    O = _bdot(h, wwide)
    return (_shift_dn(O[:, :C]) + O[:, C:2 * C]
            + _shift_up(O[:, 2 * C:]) + b)


def _vp_body(h, C, w1, b1, g1, be1, w2, b2, g2, be2, lwrow):
    h = _ln(jax.nn.relu(_conv3(h, w1, b1, C)), g1, be1)
    h = _ln(jax.nn.relu(_conv3(h, w2, b2, C)), g2, be2)
    return _row_dot(lwrow, h)


def _mega_kernel(S, T, D, C, NB,
                 x_ref, durf_ref, ptrg_ref, lob_ref, hib_ref, maxd_ref,
                 dw1, db1, dg1, dbe1, dw2, db2, dg2, dbe2, dlw,
                 pw1, pb1, pg1, pbe1, pw2, pb2, pg2, pbe2, plw,
                 ew1, eb1, eg1, ebe1, ew2, eb2, eg2, ebe2, elw,
                 tabs_ref, lbs_ref,
                 xe_ref, pemb_ref, eemb_ref, ppred_ref, epred_ref,
                 dpred_ref, maskf_ref):
    xb = x_ref[0]
    durf = durf_ref[0]
    ptrg = ptrg_ref[0]
    maxd = maxd_ref[0, 0]
    dlb = lbs_ref[0, 0]
    plb = lbs_ref[0, 1]
    elb = lbs_ref[0, 2]

    dpred = _vp_body(xb, C, dw1[...], db1[...], dg1[...], dbe1[...],
                     dw2[...], db2[...], dg2[...], dbe2[...], dlw[...])
    dpred_ref[0, 0, :] = dpred[0] + dlb

    r = jax.lax.broadcasted_iota(jnp.int32, (S, S), 0)
    c = jax.lax.broadcasted_iota(jnp.int32, (S, S), 1)
    tri = (r <= c).astype(_BF16)
    cum = _bdot(durf, tri)
    cum_prev = cum - durf
    mel_len = cum[0, S - 1]
    lim = jnp.minimum(mel_len, maxd)
    cumc = jnp.minimum(cum, lim)

    tt = jax.lax.broadcasted_iota(jnp.int32, (T, 1), 0).astype(_F32)
    E = ((cum_prev <= tt) & (tt < cumc)).astype(_BF16)
    xe0 = jnp.dot(E, xb.astype(_BF16), preferred_element_type=_F32)
    trow = jax.lax.broadcasted_iota(jnp.int32, (1, T), 1).astype(_F32)
    maskf_ref[0, 0, :] = (trow[0] >= lim).astype(_F32)

    pv = ptrg.reshape(T, 1)
    onehot = ((lob_ref[0:1, :] < pv) & (pv <= hib_ref[0:1, :])).astype(_BF16)
    embs = jnp.dot(onehot, tabs_ref[...].astype(_BF16),
                   preferred_element_type=_F32)
    pemb = embs[:, :D]
    eemb = embs[:, D:]
    pemb_ref[0] = pemb
    eemb_ref[0] = eemb

    validrow = (trow < lim).astype(_F32)

    ppred = _vp_body(xe0, C, pw1[...], pb1[...], pg1[...], pbe1[...],
                     pw2[...], pb2[...], pg2[...], pbe2[...], plw[...])
    ppred_ref[0, 0, :] = (ppred[0] + plb) * validrow[0]
    xe1 = xe0 + pemb

    epred = _vp_body(xe1, C, ew1[...], eb1[...], eg1[...], ebe1[...],
                     ew2[...], eb2[...], eg2[...], ebe2[...], elw[...])
    epred_ref[0, 0, :] = (epred[0] + elb) * validrow[0]
    xe_ref[0] = xe1 + eemb


def _wide_conv_w(w):
    K = w.shape[2]
    return jnp.concatenate([w[:, :, k].T for k in range(K)], axis=1)


def _vp_args(p):
    C = p['c1b'].shape[0]
    return (
        _wide_conv_w(p['c1w']).astype(_BF16), p['c1b'].reshape(1, C),
        p['g1'].reshape(1, C), p['b1'].reshape(1, C),
        _wide_conv_w(p['c2w']).astype(_BF16), p['c2b'].reshape(1, C),
        p['g2'].reshape(1, C), p['b2'].reshape(1, C),
        p['lw'].reshape(1, C).astype(_BF16),
    )


def kernel(x, dur_trg, pitch_trg, energy_trg, src_mask, max_dur,
           dp, pp, ep, pitch_bins, energy_bins, pitch_table, energy_table):
    B, S, D = x.shape
    T = pitch_trg.shape[1]
    C = dp['c1b'].shape[0]
    NB = pitch_table.shape[0]

    durf = dur_trg.astype(_F32).reshape(B, 1, S)
    ptrg = pitch_trg.reshape(B, 1, T)
    binsf = pitch_bins.astype(_F32)
    lob = jnp.concatenate([jnp.full((1,), -1e30, _F32), binsf]).reshape(1, NB)
    hib = jnp.concatenate([binsf, jnp.full((1,), 1e30, _F32)]).reshape(1, NB)
    lob = jnp.broadcast_to(lob, (8, NB))
    hib = jnp.broadcast_to(hib, (8, NB))
    maxd_arr = jnp.full((8, 128), max_dur, _F32)
    tabs = jnp.concatenate([pitch_table, energy_table], axis=1)
    lbs = jnp.stack([dp['lb'], pp['lb'], ep['lb']]).reshape(1, 3)
    lbs = jnp.broadcast_to(jnp.pad(lbs, ((0, 0), (0, 125))), (8, 128))

    vp_all = _vp_args(dp) + _vp_args(pp) + _vp_args(ep)

    def full(a):
        return pl.BlockSpec(a.shape, lambda b: (0,) * a.ndim)

    in_specs = (
        [pl.BlockSpec((1, S, D), lambda b: (b, 0, 0)),
         pl.BlockSpec((1, 1, S), lambda b: (b, 0, 0)),
         pl.BlockSpec((1, 1, T), lambda b: (b, 0, 0)),
         full(lob), full(hib), full(maxd_arr)]
        + [full(a) for a in vp_all]
        + [full(tabs), full(lbs)]
    )
    out_specs = [
        pl.BlockSpec((1, T, D), lambda b: (b, 0, 0)),
        pl.BlockSpec((1, T, D), lambda b: (b, 0, 0)),
        pl.BlockSpec((1, T, D), lambda b: (b, 0, 0)),
        pl.BlockSpec((1, 1, T), lambda b: (b, 0, 0)),
        pl.BlockSpec((1, 1, T), lambda b: (b, 0, 0)),
        pl.BlockSpec((1, 1, S), lambda b: (b, 0, 0)),
        pl.BlockSpec((1, 1, T), lambda b: (b, 0, 0)),
    ]
    out_shapes = [
        jax.ShapeDtypeStruct((B, T, D), _F32),
        jax.ShapeDtypeStruct((B, T, D), _F32),
        jax.ShapeDtypeStruct((B, T, D), _F32),
        jax.ShapeDtypeStruct((B, 1, T), _F32),
        jax.ShapeDtypeStruct((B, 1, T), _F32),
        jax.ShapeDtypeStruct((B, 1, S), _F32),
        jax.ShapeDtypeStruct((B, 1, T), _F32),
    ]

    body = functools.partial(_mega_kernel, S, T, D, C, NB)
    xe, pemb, eemb, ppred, epred, dpred, maskf = pl.pallas_call(
        body,
        grid=(B,),
        in_specs=in_specs,
        out_specs=out_specs,
        out_shape=out_shapes,
    )(x, durf, ptrg, lob, hib, maxd_arr, *vp_all, tabs, lbs)

    mel_mask = maskf.reshape(B, T) > 0.5
    log_dur_pred = jnp.where(src_mask, 0.0, dpred.reshape(B, S))
    return (xe, mel_mask, log_dur_pred, dur_trg,
            ppred.reshape(B, T), pemb, epred.reshape(B, T), eemb)
